# Initial kernel scaffold; baseline (speedup 1.0000x reference)
import jax
import jax.numpy as jnp
from jax.experimental import pallas as pl


def kernel(X_input, Z0, emb0, beta_1, beta_2, beta_3):
    raise NotImplementedError("write your pallas kernel here")


# speedup vs baseline: 12.1420x; 12.1420x over previous
import functools

import jax
import jax.numpy as jnp
from jax import lax
from jax.experimental import pallas as pl
from jax.experimental.pallas import tpu as pltpu
from jax.experimental.pallas import tpu_sc as plsc

N = 320000
Q = 10000
D = 128

NC = 2
NS = 16
NW = NC * NS
L = 16

ROWS = N // NW
BR = 80
NB = ROWS // BR
CH = 2000
NCH = ROWS // CH
BPC = CH // BR

QT = Q // NS

CB_W = 25
CB_R = Q // CB_W

_PARAMS = pltpu.CompilerParams(
    use_tc_tiling_on_sc=False, needs_layout_passes=False)


def _mesh():
    return plsc.VectorSubcoreMesh(
        core_axis_name="c", subcore_axis_name="s",
        num_cores=NC, num_subcores=NS)


def _lane_iota():
    return lax.iota(jnp.int32, L)


def _load_z(z_hbm, zbuf, sem, base):
    descs = []
    for j in range(NB):
        descs.append(pltpu.async_copy(
            z_hbm.at[pl.ds(base + j * BR, BR)], zbuf.at[j], sem))
    for d in descs:
        d.wait()


def _scatter_kernel_body(z_hbm, emb_hbm, out_hbm, zbuf, ebuf, stage, acc,
                         sem):
    cid = lax.axis_index("c")
    sid = lax.axis_index("s")
    wid = sid * NC + cid

    def _zrow(i, _):
        stage[i, :] = jnp.zeros((L,), jnp.float32)
        return ()

    lax.fori_loop(0, QT, _zrow, ())
    pltpu.sync_copy(stage, acc.at[pl.ds(sid * QT, QT)])
    plsc.subcore_barrier()

    row0 = wid * ROWS
    _load_z(z_hbm, zbuf, sem, row0)
    lane = _lane_iota()
    col3 = jnp.full((L,), 3, jnp.int32)
    ones = jnp.ones((L,), jnp.float32)

    def _chunk(c, _):
        pltpu.sync_copy(
            emb_hbm.at[pl.ds(row0 + c * CH, CH), pl.ds(0, 16)], ebuf)

        def _batch(b, _):
            for g in range(BR // L):
                ridx = b * BR + g * L + lane
                plsc.store_scatter(ebuf, [ridx, col3], ones)
            jb = c * BPC + b
            pltpu.sync_copy(
                ebuf.at[pl.ds(b * BR, BR)], acc.at[zbuf.at[jb]], add=True)
            return ()

        lax.fori_loop(0, BPC, _batch, ())
        return ()

    lax.fori_loop(0, NCH, _chunk, ())
    plsc.subcore_barrier()

    pltpu.sync_copy(acc.at[pl.ds(sid * QT, QT)], stage)
    pltpu.sync_copy(stage, out_hbm.at[cid, pl.ds(sid * QT, QT)])


def _combine_kernel_body(part_hbm, beta_hbm, out_hbm, p0, p1, obuf, bbuf):
    cid = lax.axis_index("c")
    sid = lax.axis_index("s")
    wid = sid * NC + cid

    @pl.when(wid < CB_W)
    def _():
        base = wid * CB_R
        pltpu.sync_copy(part_hbm.at[0, pl.ds(base, CB_R)], p0)
        pltpu.sync_copy(part_hbm.at[1, pl.ds(base, CB_R)], p1)
        pltpu.sync_copy(beta_hbm, bbuf)
        bvec = bbuf[:]
        lane = _lane_iota()

        def _row(i, _):
            srow = p0[i, :] + p1[i, :]
            cntv = jnp.full((L,), srow[3])
            zero = cntv == 0.0
            safe = jnp.where(zero, 1.0, cntv)
            meanv = jnp.where(zero, 0.0, srow / safe)
            t = bvec + meanv
            row = jnp.where(lane == 2, jnp.maximum(t, 0.1), t)
            row = jnp.where(lane < 3, row, 0.0)
            obuf[i, :] = row
            return ()

        lax.fori_loop(0, CB_R, _row, ())
        pltpu.sync_copy(obuf, out_hbm.at[pl.ds(base, CB_R)])


def _gather_kernel_body(z_hbm, x_hbm, bp_hbm, out_hbm, zbuf, xbuf, obuf,
                        gbuf, sem):
    cid = lax.axis_index("c")
    sid = lax.axis_index("s")
    wid = sid * NC + cid
    base = wid * ROWS

    _load_z(z_hbm, zbuf, sem, base)
    pltpu.sync_copy(x_hbm.at[pl.ds(base, ROWS)], xbuf)
    lane = _lane_iota()
    c0 = jnp.full((L,), 0, jnp.int32)
    c1 = jnp.full((L,), 1, jnp.int32)
    c2 = jnp.full((L,), 2, jnp.int32)

    def _batch(j, _):
        pltpu.async_copy(bp_hbm.at[zbuf.at[j]], gbuf, sem).wait()
        for g in range(BR // L):
            ridx = g * L + lane
            b0 = plsc.load_gather(gbuf, [ridx, c0])
            b1 = plsc.load_gather(gbuf, [ridx, c1])
            b2 = plsc.load_gather(gbuf, [ridx, c2])
            x = xbuf[pl.ds(j * BR + g * L, L)]
            t = (x - b1) / b2
            o = b0 / (1.0 + jnp.exp(-t))
            obuf[pl.ds(j * BR + g * L, L)] = o
        return ()

    lax.fori_loop(0, NB, _batch, ())
    pltpu.sync_copy(obuf, out_hbm.at[pl.ds(base, ROWS)])


@functools.cache
def _build_kernels():
    mesh = _mesh()
    scatter = pl.kernel(
        _scatter_kernel_body,
        out_type=jax.ShapeDtypeStruct((NC, Q, 16), jnp.float32),
        mesh=mesh,
        compiler_params=_PARAMS,
        scratch_types=[
            pltpu.VMEM((NB, BR), jnp.int32),
            pltpu.VMEM((CH, 16), jnp.float32),
            pltpu.VMEM((QT, 16), jnp.float32),
            pltpu.VMEM_SHARED((Q, 16), jnp.float32),
            pltpu.SemaphoreType.DMA,
        ],
    )
    combine = pl.kernel(
        _combine_kernel_body,
        out_type=jax.ShapeDtypeStruct((Q, 16), jnp.float32),
        mesh=mesh,
        compiler_params=_PARAMS,
        scratch_types=[
            pltpu.VMEM((CB_R, 16), jnp.float32),
            pltpu.VMEM((CB_R, 16), jnp.float32),
            pltpu.VMEM((CB_R, 16), jnp.float32),
            pltpu.VMEM((L,), jnp.float32),
        ],
    )
    gather = pl.kernel(
        _gather_kernel_body,
        out_type=jax.ShapeDtypeStruct((N,), jnp.float32),
        mesh=mesh,
        compiler_params=_PARAMS,
        scratch_types=[
            pltpu.VMEM((NB, BR), jnp.int32),
            pltpu.VMEM((ROWS,), jnp.float32),
            pltpu.VMEM((ROWS,), jnp.float32),
            pltpu.VMEM((BR, 16), jnp.float32),
            pltpu.SemaphoreType.DMA,
        ],
    )
    return scatter, combine, gather


def kernel(X_input, Z0, emb0, beta_1, beta_2, beta_3):
    scatter, combine, gather = _build_kernels()
    z = Z0.astype(jnp.int32)
    x = X_input.astype(jnp.float32).reshape(N)
    betavec = jnp.concatenate([
        jnp.stack([beta_1, beta_2, beta_3]).astype(jnp.float32),
        jnp.zeros((13,), jnp.float32),
    ])
    partials = scatter(z, emb0)
    bp = combine(partials, betavec)
    out = gather(z, x, bp)
    return out.reshape(N, 1)


# speedup vs baseline: 25.0343x; 2.0618x over previous
import functools

import jax
import jax.numpy as jnp
from jax import lax
from jax.experimental import pallas as pl
from jax.experimental.pallas import tpu as pltpu
from jax.experimental.pallas import tpu_sc as plsc

N = 320000
Q = 10000
D = 128

NC = 2
NS = 16
NW = NC * NS
L = 16

ROWS = N // NW
BR = 80
NB = ROWS // BR
CH = 2000
NCH = ROWS // CH
BPC = CH // BR

QT = Q // NS

CB_W = 25
CB_R = Q // CB_W
CB_G = CB_R // L

_PARAMS = pltpu.CompilerParams(
    use_tc_tiling_on_sc=False, needs_layout_passes=False)


def _mesh():
    return plsc.VectorSubcoreMesh(
        core_axis_name="c", subcore_axis_name="s",
        num_cores=NC, num_subcores=NS)


def _lane_iota():
    return lax.iota(jnp.int32, L)


def _scatter_kernel_body(z_hbm, emb_hbm, out_hbm, zbuf, eb0, eb1, stage,
                         acc, sem_z, sem_e, sem_s):
    cid = lax.axis_index("c")
    sid = lax.axis_index("s")
    wid = sid * NC + cid
    row0 = wid * ROWS

    zdescs = [
        pltpu.async_copy(
            z_hbm.at[pl.ds(row0 + j * BR, BR)], zbuf.at[j], sem_z)
        for j in range(NB)
    ]

    def _zrow(i, _):
        stage[i, :] = jnp.zeros((L,), jnp.float32)
        return ()

    lax.fori_loop(0, QT, _zrow, ())
    pltpu.sync_copy(stage, acc.at[pl.ds(sid * QT, QT)])
    plsc.subcore_barrier()
    for d in zdescs:
        d.wait()

    lane = _lane_iota()
    col3 = jnp.full((L,), 3, jnp.int32)
    ones = jnp.ones((L,), jnp.float32)
    ebufs = [eb0, eb1]

    def _emb_load(c, buf):
        return pltpu.async_copy(
            emb_hbm.at[pl.ds(row0 + c * CH, CH), pl.ds(0, 16)], buf, sem_e)

    pend_e = _emb_load(0, ebufs[0])
    for c in range(NCH):
        pend_e.wait()
        if c + 1 < NCH:
            pend_e = _emb_load(c + 1, ebufs[(c + 1) % 2])
        ebuf = ebufs[c % 2]
        sdescs = []
        for b in range(BPC):
            for g in range(BR // L):
                ridx = b * BR + g * L + lane
                plsc.store_scatter(ebuf, [ridx, col3], ones)
            jb = c * BPC + b
            sdescs.append(pltpu.async_copy(
                ebuf.at[pl.ds(b * BR, BR)], acc.at[zbuf.at[jb]], sem_s,
                add=True))
        for d in sdescs:
            d.wait()

    plsc.subcore_barrier()

    pltpu.sync_copy(acc.at[pl.ds(sid * QT, QT)], stage)
    pltpu.sync_copy(stage, out_hbm.at[cid, pl.ds(sid * QT, QT)])


def _combine_kernel_body(part_hbm, beta_hbm, out_hbm, p0, p1, o0, o1, o2,
                         bbuf):
    cid = lax.axis_index("c")
    sid = lax.axis_index("s")
    wid = sid * NC + cid

    @pl.when(wid < CB_W)
    def _():
        base = wid * CB_R
        pltpu.sync_copy(part_hbm.at[0, pl.ds(base, CB_R)], p0)
        pltpu.sync_copy(part_hbm.at[1, pl.ds(base, CB_R)], p1)
        pltpu.sync_copy(beta_hbm, bbuf)
        bvec = bbuf[:]
        b1 = bvec[0]
        b2 = bvec[1]
        b3 = bvec[2]
        lane = _lane_iota()
        c0 = jnp.full((L,), 0, jnp.int32)
        c1 = jnp.full((L,), 1, jnp.int32)
        c2 = jnp.full((L,), 2, jnp.int32)
        c3 = jnp.full((L,), 3, jnp.int32)

        def _group(i, _):
            qidx = i * L + lane
            s0 = (plsc.load_gather(p0, [qidx, c0])
                  + plsc.load_gather(p1, [qidx, c0]))
            s1 = (plsc.load_gather(p0, [qidx, c1])
                  + plsc.load_gather(p1, [qidx, c1]))
            s2 = (plsc.load_gather(p0, [qidx, c2])
                  + plsc.load_gather(p1, [qidx, c2]))
            cnt = (plsc.load_gather(p0, [qidx, c3])
                   + plsc.load_gather(p1, [qidx, c3]))
            zero = cnt == 0.0
            safe = jnp.where(zero, 1.0, cnt)
            m0 = jnp.where(zero, 0.0, s0 / safe)
            m1 = jnp.where(zero, 0.0, s1 / safe)
            m2 = jnp.where(zero, 0.0, s2 / safe)
            o0[pl.ds(i * L, L)] = b1 + m0
            o1[pl.ds(i * L, L)] = b2 + m1
            o2[pl.ds(i * L, L)] = jnp.maximum(b3 + m2, 0.1)
            return ()

        lax.fori_loop(0, CB_G, _group, ())
        pltpu.sync_copy(o0, out_hbm.at[0, pl.ds(base, CB_R)])
        pltpu.sync_copy(o1, out_hbm.at[1, pl.ds(base, CB_R)])
        pltpu.sync_copy(o2, out_hbm.at[2, pl.ds(base, CB_R)])


def _gather_kernel_body(z_hbm, x_hbm, bp_hbm, out_hbm, zbuf, xbuf, obuf,
                        t0, t1, t2, sem):
    cid = lax.axis_index("c")
    sid = lax.axis_index("s")
    wid = sid * NC + cid
    base = wid * ROWS

    descs = [
        pltpu.async_copy(bp_hbm.at[0], t0, sem),
        pltpu.async_copy(bp_hbm.at[1], t1, sem),
        pltpu.async_copy(bp_hbm.at[2], t2, sem),
        pltpu.async_copy(z_hbm.at[pl.ds(base, ROWS)], zbuf, sem),
        pltpu.async_copy(x_hbm.at[pl.ds(base, ROWS)], xbuf, sem),
    ]
    for d in descs:
        d.wait()

    def _block(j, _):
        for g in range(5):
            off = j * BR + g * L
            zvec = zbuf[pl.ds(off, L)]
            b0 = plsc.load_gather(t0, [zvec])
            b1 = plsc.load_gather(t1, [zvec])
            b2 = plsc.load_gather(t2, [zvec])
            x = xbuf[pl.ds(off, L)]
            t = (x - b1) / b2
            obuf[pl.ds(off, L)] = b0 / (1.0 + jnp.exp(-t))
        return ()

    lax.fori_loop(0, NB, _block, ())
    pltpu.sync_copy(obuf, out_hbm.at[pl.ds(base, ROWS)])


@functools.cache
def _build_kernels():
    mesh = _mesh()
    scatter = pl.kernel(
        _scatter_kernel_body,
        out_type=jax.ShapeDtypeStruct((NC, Q, 16), jnp.float32),
        mesh=mesh,
        compiler_params=_PARAMS,
        scratch_types=[
            pltpu.VMEM((NB, BR), jnp.int32),
            pltpu.VMEM((CH, 16), jnp.float32),
            pltpu.VMEM((CH, 16), jnp.float32),
            pltpu.VMEM((QT, 16), jnp.float32),
            pltpu.VMEM_SHARED((Q, 16), jnp.float32),
            pltpu.SemaphoreType.DMA,
            pltpu.SemaphoreType.DMA,
            pltpu.SemaphoreType.DMA,
        ],
    )
    combine = pl.kernel(
        _combine_kernel_body,
        out_type=jax.ShapeDtypeStruct((3, Q), jnp.float32),
        mesh=mesh,
        compiler_params=_PARAMS,
        scratch_types=[
            pltpu.VMEM((CB_R, 16), jnp.float32),
            pltpu.VMEM((CB_R, 16), jnp.float32),
            pltpu.VMEM((CB_R,), jnp.float32),
            pltpu.VMEM((CB_R,), jnp.float32),
            pltpu.VMEM((CB_R,), jnp.float32),
            pltpu.VMEM((L,), jnp.float32),
        ],
    )
    gather = pl.kernel(
        _gather_kernel_body,
        out_type=jax.ShapeDtypeStruct((N,), jnp.float32),
        mesh=mesh,
        compiler_params=_PARAMS,
        scratch_types=[
            pltpu.VMEM((ROWS,), jnp.int32),
            pltpu.VMEM((ROWS,), jnp.float32),
            pltpu.VMEM((ROWS,), jnp.float32),
            pltpu.VMEM((Q,), jnp.float32),
            pltpu.VMEM((Q,), jnp.float32),
            pltpu.VMEM((Q,), jnp.float32),
            pltpu.SemaphoreType.DMA,
        ],
    )
    return scatter, combine, gather


def kernel(X_input, Z0, emb0, beta_1, beta_2, beta_3):
    scatter, combine, gather = _build_kernels()
    z = Z0.astype(jnp.int32)
    x = X_input.astype(jnp.float32).reshape(N)
    betavec = jnp.concatenate([
        jnp.stack([beta_1, beta_2, beta_3]).astype(jnp.float32),
        jnp.zeros((13,), jnp.float32),
    ])
    partials = scatter(z, emb0)
    bp = combine(partials, betavec)
    out = gather(z, x, bp)
    return out.reshape(N, 1)


# speedup vs baseline: 28.3178x; 1.1312x over previous
import functools

import jax
import jax.numpy as jnp
from jax import lax
from jax.experimental import pallas as pl
from jax.experimental.pallas import tpu as pltpu
from jax.experimental.pallas import tpu_sc as plsc

N = 320000
Q = 10000
D = 128

NC = 2
NS = 16
NW = NC * NS
L = 16

ROWS = N // NW
BR = 80
NB = ROWS // BR
CH = 2000
NCH = ROWS // CH
BPC = CH // BR

QT = Q // NS

CB_G = 39
CB_R = CB_G * L
CB_PAD = CB_R + L

_PARAMS = pltpu.CompilerParams(
    use_tc_tiling_on_sc=False, needs_layout_passes=False)


def _mesh():
    return plsc.VectorSubcoreMesh(
        core_axis_name="c", subcore_axis_name="s",
        num_cores=NC, num_subcores=NS)


def _lane_iota():
    return lax.iota(jnp.int32, L)


def _scatter_kernel_body(z_hbm, emb_hbm, out_hbm, zbuf, eb0, eb1, stage,
                         acc, sem_z, sem_e, sem_s):
    cid = lax.axis_index("c")
    sid = lax.axis_index("s")
    wid = sid * NC + cid
    row0 = wid * ROWS

    zdescs = [
        pltpu.async_copy(
            z_hbm.at[pl.ds(row0 + j * BR, BR)], zbuf.at[j], sem_z)
        for j in range(NB)
    ]

    def _zrow(i, _):
        stage[i, :] = jnp.zeros((L,), jnp.float32)
        return ()

    lax.fori_loop(0, QT, _zrow, ())
    pltpu.sync_copy(stage, acc.at[pl.ds(sid * QT, QT)])
    plsc.subcore_barrier()
    for d in zdescs:
        d.wait()

    lane = _lane_iota()
    col3 = jnp.full((L,), 3, jnp.int32)
    ones = jnp.ones((L,), jnp.float32)
    ebufs = [eb0, eb1]

    def _emb_load(c, buf):
        return pltpu.async_copy(
            emb_hbm.at[pl.ds(row0 + c * CH, CH), pl.ds(0, 16)], buf, sem_e)

    pend_e = _emb_load(0, ebufs[0])
    for c in range(NCH):
        pend_e.wait()
        if c + 1 < NCH:
            pend_e = _emb_load(c + 1, ebufs[(c + 1) % 2])
        ebuf = ebufs[c % 2]
        sdescs = []
        for b in range(BPC):
            for g in range(BR // L):
                ridx = b * BR + g * L + lane
                plsc.store_scatter(ebuf, [ridx, col3], ones)
            jb = c * BPC + b
            sdescs.append(pltpu.async_copy(
                ebuf.at[pl.ds(b * BR, BR)], acc.at[zbuf.at[jb]], sem_s,
                add=True))
        for d in sdescs:
            d.wait()

    plsc.subcore_barrier()

    pltpu.sync_copy(acc.at[pl.ds(sid * QT, QT)], stage)
    pltpu.sync_copy(stage, out_hbm.at[cid, pl.ds(sid * QT, QT)])


def _fused_kernel_body(z_hbm, x_hbm, part_hbm, beta_hbm, out_hbm,
                       zbuf, xbuf, obuf, t0, t1, t2, p0, p1,
                       ob0, ob1, ob2, bbuf, planes, sem):
    cid = lax.axis_index("c")
    sid = lax.axis_index("s")
    wid = sid * NC + cid
    base = wid * ROWS

    dz = pltpu.async_copy(z_hbm.at[pl.ds(base, ROWS)], zbuf, sem)
    dx = pltpu.async_copy(x_hbm.at[pl.ds(base, ROWS)], xbuf, sem)

    seg0 = sid * CB_R
    pltpu.sync_copy(part_hbm.at[0, pl.ds(seg0, CB_R)],
                    p0.at[pl.ds(0, CB_R)])
    pltpu.sync_copy(part_hbm.at[1, pl.ds(seg0, CB_R)],
                    p1.at[pl.ds(0, CB_R)])

    @pl.when(sid == 0)
    def _():
        pltpu.sync_copy(part_hbm.at[0, pl.ds(NS * CB_R, L)],
                        p0.at[pl.ds(CB_R, L)])
        pltpu.sync_copy(part_hbm.at[1, pl.ds(NS * CB_R, L)],
                        p1.at[pl.ds(CB_R, L)])

    pltpu.sync_copy(beta_hbm, bbuf)
    bvec = bbuf[:]
    b1s = bvec[0]
    b2s = bvec[1]
    b3s = bvec[2]
    lane = _lane_iota()
    c0 = jnp.full((L,), 0, jnp.int32)
    c1 = jnp.full((L,), 1, jnp.int32)
    c2 = jnp.full((L,), 2, jnp.int32)
    c3 = jnp.full((L,), 3, jnp.int32)

    def _combine_group(i):
        qidx = i * L + lane
        s0 = (plsc.load_gather(p0, [qidx, c0])
              + plsc.load_gather(p1, [qidx, c0]))
        s1 = (plsc.load_gather(p0, [qidx, c1])
              + plsc.load_gather(p1, [qidx, c1]))
        s2 = (plsc.load_gather(p0, [qidx, c2])
              + plsc.load_gather(p1, [qidx, c2]))
        cnt = (plsc.load_gather(p0, [qidx, c3])
               + plsc.load_gather(p1, [qidx, c3]))
        zero = cnt == 0.0
        safe = jnp.where(zero, 1.0, cnt)
        m0 = jnp.where(zero, 0.0, s0 / safe)
        m1 = jnp.where(zero, 0.0, s1 / safe)
        m2 = jnp.where(zero, 0.0, s2 / safe)
        ob0[pl.ds(i * L, L)] = b1s + m0
        ob1[pl.ds(i * L, L)] = b2s + m1
        ob2[pl.ds(i * L, L)] = jnp.maximum(b3s + m2, 0.1)

    def _cg(i, _):
        _combine_group(i)
        return ()

    lax.fori_loop(0, CB_G, _cg, ())

    pltpu.sync_copy(ob0.at[pl.ds(0, CB_R)],
                    planes.at[0, pl.ds(seg0, CB_R)])
    pltpu.sync_copy(ob1.at[pl.ds(0, CB_R)],
                    planes.at[1, pl.ds(seg0, CB_R)])
    pltpu.sync_copy(ob2.at[pl.ds(0, CB_R)],
                    planes.at[2, pl.ds(seg0, CB_R)])

    @pl.when(sid == 0)
    def _():
        _combine_group(CB_G)
        pltpu.sync_copy(ob0.at[pl.ds(CB_R, L)],
                        planes.at[0, pl.ds(NS * CB_R, L)])
        pltpu.sync_copy(ob1.at[pl.ds(CB_R, L)],
                        planes.at[1, pl.ds(NS * CB_R, L)])
        pltpu.sync_copy(ob2.at[pl.ds(CB_R, L)],
                        planes.at[2, pl.ds(NS * CB_R, L)])

    plsc.subcore_barrier()

    pltpu.sync_copy(planes.at[0], t0)
    pltpu.sync_copy(planes.at[1], t1)
    pltpu.sync_copy(planes.at[2], t2)
    dz.wait()
    dx.wait()

    def _block(j, _):
        for g in range(5):
            off = j * BR + g * L
            zvec = zbuf[pl.ds(off, L)]
            b0 = plsc.load_gather(t0, [zvec])
            b1 = plsc.load_gather(t1, [zvec])
            b2 = plsc.load_gather(t2, [zvec])
            x = xbuf[pl.ds(off, L)]
            t = (x - b1) / b2
            obuf[pl.ds(off, L)] = b0 / (1.0 + jnp.exp(-t))
        return ()

    lax.fori_loop(0, NB, _block, ())
    pltpu.sync_copy(obuf, out_hbm.at[pl.ds(base, ROWS)])


@functools.cache
def _build_kernels():
    mesh = _mesh()
    scatter = pl.kernel(
        _scatter_kernel_body,
        out_type=jax.ShapeDtypeStruct((NC, Q, 16), jnp.float32),
        mesh=mesh,
        compiler_params=_PARAMS,
        scratch_types=[
            pltpu.VMEM((NB, BR), jnp.int32),
            pltpu.VMEM((CH, 16), jnp.float32),
            pltpu.VMEM((CH, 16), jnp.float32),
            pltpu.VMEM((QT, 16), jnp.float32),
            pltpu.VMEM_SHARED((Q, 16), jnp.float32),
            pltpu.SemaphoreType.DMA,
            pltpu.SemaphoreType.DMA,
            pltpu.SemaphoreType.DMA,
        ],
    )
    fused = pl.kernel(
        _fused_kernel_body,
        out_type=jax.ShapeDtypeStruct((N,), jnp.float32),
        mesh=mesh,
        compiler_params=_PARAMS,
        scratch_types=[
            pltpu.VMEM((ROWS,), jnp.int32),
            pltpu.VMEM((ROWS,), jnp.float32),
            pltpu.VMEM((ROWS,), jnp.float32),
            pltpu.VMEM((Q,), jnp.float32),
            pltpu.VMEM((Q,), jnp.float32),
            pltpu.VMEM((Q,), jnp.float32),
            pltpu.VMEM((CB_PAD, 16), jnp.float32),
            pltpu.VMEM((CB_PAD, 16), jnp.float32),
            pltpu.VMEM((CB_PAD,), jnp.float32),
            pltpu.VMEM((CB_PAD,), jnp.float32),
            pltpu.VMEM((CB_PAD,), jnp.float32),
            pltpu.VMEM((L,), jnp.float32),
            pltpu.VMEM_SHARED((3, Q), jnp.float32),
            pltpu.SemaphoreType.DMA,
        ],
    )
    return scatter, fused


def kernel(X_input, Z0, emb0, beta_1, beta_2, beta_3):
    scatter, fused = _build_kernels()
    z = Z0.astype(jnp.int32)
    x = X_input.astype(jnp.float32).reshape(N)
    betavec = jnp.concatenate([
        jnp.stack([beta_1, beta_2, beta_3]).astype(jnp.float32),
        jnp.zeros((13,), jnp.float32),
    ])
    partials = scatter(z, emb0)
    out = fused(z, x, partials, betavec)
    return out.reshape(N, 1)


# speedup vs baseline: 29.4427x; 1.0397x over previous
import functools

import jax
import jax.numpy as jnp
from jax import lax
from jax.experimental import pallas as pl
from jax.experimental.pallas import tpu as pltpu
from jax.experimental.pallas import tpu_sc as plsc

N = 320000
Q = 10000
D = 128

NC = 2
NS = 16
NW = NC * NS
L = 16

ROWS = N // NW
BR = 80
NB = ROWS // BR
CH = 2000
NCH = ROWS // CH
BPC = CH // BR

QT = Q // NS

CB_G = 39
CB_R = CB_G * L
CB_PAD = CB_R + L

_PARAMS = pltpu.CompilerParams(
    use_tc_tiling_on_sc=False, needs_layout_passes=False)


def _mesh():
    return plsc.VectorSubcoreMesh(
        core_axis_name="c", subcore_axis_name="s",
        num_cores=NC, num_subcores=NS)


def _lane_iota():
    return lax.iota(jnp.int32, L)


def _scatter_kernel_body(z_hbm, emb_hbm, out_hbm, zbuf, eb0, eb1, stage,
                         acc, sem_z, sem_e, sem_s):
    cid = lax.axis_index("c")
    sid = lax.axis_index("s")
    wid = sid * NC + cid
    row0 = wid * ROWS

    zdescs = [
        pltpu.async_copy(
            z_hbm.at[pl.ds(row0 + j * BR, BR)], zbuf.at[j], sem_z)
        for j in range(NB)
    ]

    def _zrow(i, _):
        stage[i, :] = jnp.zeros((L,), jnp.float32)
        return ()

    lax.fori_loop(0, QT, _zrow, ())
    pltpu.sync_copy(stage, acc.at[pl.ds(sid * QT, QT)])
    plsc.subcore_barrier()
    for d in zdescs:
        d.wait()

    lane = _lane_iota()
    col3 = jnp.full((L,), 3, jnp.int32)
    ones = jnp.ones((L,), jnp.float32)
    ebufs = [eb0, eb1]

    def _emb_load(c, buf):
        return pltpu.async_copy(
            emb_hbm.at[pl.ds(row0 + c * CH, CH), pl.ds(0, 16)], buf, sem_e)

    pend_e = _emb_load(0, ebufs[0])
    for c in range(NCH):
        pend_e.wait()
        if c + 1 < NCH:
            pend_e = _emb_load(c + 1, ebufs[(c + 1) % 2])
        ebuf = ebufs[c % 2]
        sdescs = []
        for b in range(BPC):
            for g in range(BR // L):
                ridx = b * BR + g * L + lane
                plsc.store_scatter(ebuf, [ridx, col3], ones)
            jb = c * BPC + b
            sdescs.append(pltpu.async_copy(
                ebuf.at[pl.ds(b * BR, BR)], acc.at[zbuf.at[jb]], sem_s,
                add=True))
        for d in sdescs:
            d.wait()

    plsc.subcore_barrier()

    pltpu.sync_copy(acc.at[pl.ds(sid * QT, QT)], stage)
    pltpu.sync_copy(stage, out_hbm.at[cid, pl.ds(sid * QT, QT)])


def _fused_kernel_body(z_hbm, x_hbm, part_hbm, beta_hbm, out_hbm,
                       zbuf, xbuf, obuf, t0, t1, t2, p0, p1,
                       ob0, ob1, ob2, bbuf, planes, sem):
    cid = lax.axis_index("c")
    sid = lax.axis_index("s")
    wid = sid * NC + cid
    base = wid * ROWS

    dz = pltpu.async_copy(z_hbm.at[pl.ds(base, ROWS)], zbuf, sem)
    dx = pltpu.async_copy(x_hbm.at[pl.ds(base, ROWS)], xbuf, sem)

    seg0 = sid * CB_R
    pltpu.sync_copy(part_hbm.at[0, pl.ds(seg0, CB_R)],
                    p0.at[pl.ds(0, CB_R)])
    pltpu.sync_copy(part_hbm.at[1, pl.ds(seg0, CB_R)],
                    p1.at[pl.ds(0, CB_R)])

    @pl.when(sid == 0)
    def _():
        pltpu.sync_copy(part_hbm.at[0, pl.ds(NS * CB_R, L)],
                        p0.at[pl.ds(CB_R, L)])
        pltpu.sync_copy(part_hbm.at[1, pl.ds(NS * CB_R, L)],
                        p1.at[pl.ds(CB_R, L)])

    pltpu.sync_copy(beta_hbm, bbuf)
    bvec = bbuf[:]
    b1s = bvec[0]
    b2s = bvec[1]
    b3s = bvec[2]
    lane = _lane_iota()
    c0 = jnp.full((L,), 0, jnp.int32)
    c1 = jnp.full((L,), 1, jnp.int32)
    c2 = jnp.full((L,), 2, jnp.int32)
    c3 = jnp.full((L,), 3, jnp.int32)

    def _combine_group(i):
        qidx = i * L + lane
        s0 = (plsc.load_gather(p0, [qidx, c0])
              + plsc.load_gather(p1, [qidx, c0]))
        s1 = (plsc.load_gather(p0, [qidx, c1])
              + plsc.load_gather(p1, [qidx, c1]))
        s2 = (plsc.load_gather(p0, [qidx, c2])
              + plsc.load_gather(p1, [qidx, c2]))
        cnt = (plsc.load_gather(p0, [qidx, c3])
               + plsc.load_gather(p1, [qidx, c3]))
        zero = cnt == 0.0
        safe = jnp.where(zero, 1.0, cnt)
        m0 = jnp.where(zero, 0.0, s0 / safe)
        m1 = jnp.where(zero, 0.0, s1 / safe)
        m2 = jnp.where(zero, 0.0, s2 / safe)
        ob0[pl.ds(i * L, L)] = b1s + m0
        ob1[pl.ds(i * L, L)] = b2s + m1
        ob2[pl.ds(i * L, L)] = -1.0 / jnp.maximum(b3s + m2, 0.1)

    def _cg(i, _):
        _combine_group(i)
        return ()

    lax.fori_loop(0, CB_G, _cg, ())

    pltpu.sync_copy(ob0.at[pl.ds(0, CB_R)],
                    planes.at[0, pl.ds(seg0, CB_R)])
    pltpu.sync_copy(ob1.at[pl.ds(0, CB_R)],
                    planes.at[1, pl.ds(seg0, CB_R)])
    pltpu.sync_copy(ob2.at[pl.ds(0, CB_R)],
                    planes.at[2, pl.ds(seg0, CB_R)])

    @pl.when(sid == 0)
    def _():
        _combine_group(CB_G)
        pltpu.sync_copy(ob0.at[pl.ds(CB_R, L)],
                        planes.at[0, pl.ds(NS * CB_R, L)])
        pltpu.sync_copy(ob1.at[pl.ds(CB_R, L)],
                        planes.at[1, pl.ds(NS * CB_R, L)])
        pltpu.sync_copy(ob2.at[pl.ds(CB_R, L)],
                        planes.at[2, pl.ds(NS * CB_R, L)])

    plsc.subcore_barrier()

    pltpu.sync_copy(planes.at[0], t0)
    pltpu.sync_copy(planes.at[1], t1)
    pltpu.sync_copy(planes.at[2], t2)
    dz.wait()
    dx.wait()

    def _block(j, _):
        for g in range(5):
            off = j * BR + g * L
            zvec = zbuf[pl.ds(off, L)]
            b0 = plsc.load_gather(t0, [zvec])
            b1 = plsc.load_gather(t1, [zvec])
            nrec = plsc.load_gather(t2, [zvec])
            x = xbuf[pl.ds(off, L)]
            obuf[pl.ds(off, L)] = b0 / (1.0 + jnp.exp((x - b1) * nrec))
        return ()

    lax.fori_loop(0, NB, _block, ())
    pltpu.sync_copy(obuf, out_hbm.at[pl.ds(base, ROWS)])


@functools.cache
def _build_kernels():
    mesh = _mesh()
    scatter = pl.kernel(
        _scatter_kernel_body,
        out_type=jax.ShapeDtypeStruct((NC, Q, 16), jnp.float32),
        mesh=mesh,
        compiler_params=_PARAMS,
        scratch_types=[
            pltpu.VMEM((NB, BR), jnp.int32),
            pltpu.VMEM((CH, 16), jnp.float32),
            pltpu.VMEM((CH, 16), jnp.float32),
            pltpu.VMEM((QT, 16), jnp.float32),
            pltpu.VMEM_SHARED((Q, 16), jnp.float32),
            pltpu.SemaphoreType.DMA,
            pltpu.SemaphoreType.DMA,
            pltpu.SemaphoreType.DMA,
        ],
    )
    fused = pl.kernel(
        _fused_kernel_body,
        out_type=jax.ShapeDtypeStruct((N,), jnp.float32),
        mesh=mesh,
        compiler_params=_PARAMS,
        scratch_types=[
            pltpu.VMEM((ROWS,), jnp.int32),
            pltpu.VMEM((ROWS,), jnp.float32),
            pltpu.VMEM((ROWS,), jnp.float32),
            pltpu.VMEM((Q,), jnp.float32),
            pltpu.VMEM((Q,), jnp.float32),
            pltpu.VMEM((Q,), jnp.float32),
            pltpu.VMEM((CB_PAD, 16), jnp.float32),
            pltpu.VMEM((CB_PAD, 16), jnp.float32),
            pltpu.VMEM((CB_PAD,), jnp.float32),
            pltpu.VMEM((CB_PAD,), jnp.float32),
            pltpu.VMEM((CB_PAD,), jnp.float32),
            pltpu.VMEM((L,), jnp.float32),
            pltpu.VMEM_SHARED((3, Q), jnp.float32),
            pltpu.SemaphoreType.DMA,
        ],
    )
    return scatter, fused


def kernel(X_input, Z0, emb0, beta_1, beta_2, beta_3):
    scatter, fused = _build_kernels()
    z = Z0.astype(jnp.int32)
    x = X_input.astype(jnp.float32).reshape(N)
    betavec = jnp.concatenate([
        jnp.stack([beta_1, beta_2, beta_3]).astype(jnp.float32),
        jnp.zeros((13,), jnp.float32),
    ])
    partials = scatter(z, emb0)
    out = fused(z, x, partials, betavec)
    return out.reshape(N, 1)


# speedup vs baseline: 31.2079x; 1.0600x over previous
import functools

import jax
import jax.numpy as jnp
from jax import lax
from jax.experimental import pallas as pl
from jax.experimental.pallas import tpu as pltpu
from jax.experimental.pallas import tpu_sc as plsc

N = 320000
Q = 10000
D = 128

NC = 2
NS = 16
NW = NC * NS
L = 16

ROWS = N // NW
CH = 2000
NCH = ROWS // CH
GPC = CH // L
QR = Q // L
QRP = 640

CB_G = 39
CB_R = CB_G * L
CB_PAD = CB_R + L

_PARAMS = pltpu.CompilerParams(
    use_tc_tiling_on_sc=False, needs_layout_passes=False)


def _mesh():
    return plsc.VectorSubcoreMesh(
        core_axis_name="c", subcore_axis_name="s",
        num_cores=NC, num_subcores=NS)


def _lane_iota():
    return lax.iota(jnp.int32, L)


def _scatter_kernel_body(z_hbm, emb_hbm, out_hbm,
                         zbuf, eb0, eb1, t0, t1, t2, t3, idxb,
                         acc0, acc1, acc2, acc3, sem_z, sem_e, sem_s):
    cid = lax.axis_index("c")
    sid = lax.axis_index("s")
    wid = sid * NC + cid
    row0 = wid * ROWS

    dz = pltpu.async_copy(z_hbm.at[pl.ds(row0, ROWS)], zbuf, sem_z)

    ebufs = [eb0, eb1]

    def _emb_load(c, buf):
        return pltpu.async_copy(
            emb_hbm.at[pl.ds(row0 + c * CH, CH), pl.ds(0, 16)], buf, sem_e)

    pend_e = _emb_load(0, ebufs[0])

    z16 = jnp.zeros((L,), jnp.float32)
    lane = _lane_iota()

    def _zrow(i, _):
        t0[i, :] = z16
        t1[i, :] = z16
        t2[i, :] = z16
        t3[i, :] = z16
        return ()

    lax.fori_loop(0, QR, _zrow, ())

    def _irow(i, _):
        idxb[pl.ds(i * L, L)] = i * L + lane
        return ()

    lax.fori_loop(0, QRP // L, _irow, ())

    accs = [acc0, acc1, acc2, acc3]
    tplanes = [t0, t1, t2, t3]
    for k in range(4):
        @pl.when(sid == k)
        def _(k=k):
            pltpu.sync_copy(tplanes[k], accs[k])

    c0 = jnp.full((L,), 0, jnp.int32)
    c1 = jnp.full((L,), 1, jnp.int32)
    c2 = jnp.full((L,), 2, jnp.int32)
    ones = jnp.ones((L,), jnp.float32)

    dz.wait()
    for c in range(NCH):
        pend_e.wait()
        if c + 1 < NCH:
            pend_e = _emb_load(c + 1, ebufs[(c + 1) % 2])
        ebuf = ebufs[c % 2]

        def _grp(i, _):
            off = i * L
            zvec = zbuf[pl.ds(c * CH + off, L)]
            zhi = jnp.right_shift(zvec, 4)
            zlo = jnp.bitwise_and(zvec, 15)
            ridx = off + lane
            e0 = plsc.load_gather(ebuf, [ridx, c0])
            e1 = plsc.load_gather(ebuf, [ridx, c1])
            e2 = plsc.load_gather(ebuf, [ridx, c2])
            plsc.addupdate_scatter(t0, [zhi, zlo], e0)
            plsc.addupdate_scatter(t1, [zhi, zlo], e1)
            plsc.addupdate_scatter(t2, [zhi, zlo], e2)
            plsc.addupdate_scatter(t3, [zhi, zlo], ones)
            return ()

        lax.fori_loop(0, GPC, _grp, ())

    plsc.subcore_barrier()

    ident = idxb.at[pl.ds(0, QR)]
    d0 = pltpu.async_copy(t0, acc0.at[ident], sem_s, add=True)
    d1 = pltpu.async_copy(t1, acc1.at[ident], sem_s, add=True)
    d2 = pltpu.async_copy(t2, acc2.at[ident], sem_s, add=True)
    d3 = pltpu.async_copy(t3, acc3.at[ident], sem_s, add=True)
    d0.wait()
    d1.wait()
    d2.wait()
    d3.wait()

    plsc.subcore_barrier()

    for k in range(4):
        @pl.when(sid == k)
        def _(k=k):
            pltpu.sync_copy(accs[k], out_hbm.at[cid, k])


def _fused_kernel_body(z_hbm, x_hbm, part_hbm, beta_hbm, out_hbm,
                       zbuf, xbuf, obuf, t0, t1, t2, pbuf,
                       ob0, ob1, ob2, bbuf, planes, sem):
    cid = lax.axis_index("c")
    sid = lax.axis_index("s")
    wid = sid * NC + cid
    base = wid * ROWS

    dz = pltpu.async_copy(z_hbm.at[pl.ds(base, ROWS)], zbuf, sem)
    dx = pltpu.async_copy(x_hbm.at[pl.ds(base, ROWS)], xbuf, sem)

    seg0 = sid * CB_R
    for c in range(NC):
        for k in range(4):
            pltpu.sync_copy(part_hbm.at[c, k, pl.ds(seg0, CB_R)],
                            pbuf.at[c * 4 + k, pl.ds(0, CB_R)])

    @pl.when(sid == 0)
    def _():
        for c in range(NC):
            for k in range(4):
                pltpu.sync_copy(part_hbm.at[c, k, pl.ds(NS * CB_R, L)],
                                pbuf.at[c * 4 + k, pl.ds(CB_R, L)])

    pltpu.sync_copy(beta_hbm, bbuf)
    bvec = bbuf[:]
    b1s = bvec[0]
    b2s = bvec[1]
    b3s = bvec[2]

    def _combine_group(i):
        sl = pl.ds(i * L, L)
        s0 = pbuf[0, sl] + pbuf[4, sl]
        s1 = pbuf[1, sl] + pbuf[5, sl]
        s2 = pbuf[2, sl] + pbuf[6, sl]
        cnt = pbuf[3, sl] + pbuf[7, sl]
        zero = cnt == 0.0
        safe = jnp.where(zero, 1.0, cnt)
        m0 = jnp.where(zero, 0.0, s0 / safe)
        m1 = jnp.where(zero, 0.0, s1 / safe)
        m2 = jnp.where(zero, 0.0, s2 / safe)
        ob0[sl] = b1s + m0
        ob1[sl] = b2s + m1
        ob2[sl] = -1.0 / jnp.maximum(b3s + m2, 0.1)

    def _cg(i, _):
        _combine_group(i)
        return ()

    lax.fori_loop(0, CB_G, _cg, ())

    pltpu.sync_copy(ob0.at[pl.ds(0, CB_R)],
                    planes.at[0, pl.ds(seg0, CB_R)])
    pltpu.sync_copy(ob1.at[pl.ds(0, CB_R)],
                    planes.at[1, pl.ds(seg0, CB_R)])
    pltpu.sync_copy(ob2.at[pl.ds(0, CB_R)],
                    planes.at[2, pl.ds(seg0, CB_R)])

    @pl.when(sid == 0)
    def _():
        _combine_group(CB_G)
        pltpu.sync_copy(ob0.at[pl.ds(CB_R, L)],
                        planes.at[0, pl.ds(NS * CB_R, L)])
        pltpu.sync_copy(ob1.at[pl.ds(CB_R, L)],
                        planes.at[1, pl.ds(NS * CB_R, L)])
        pltpu.sync_copy(ob2.at[pl.ds(CB_R, L)],
                        planes.at[2, pl.ds(NS * CB_R, L)])

    plsc.subcore_barrier()

    pltpu.sync_copy(planes.at[0], t0)
    pltpu.sync_copy(planes.at[1], t1)
    pltpu.sync_copy(planes.at[2], t2)
    dz.wait()
    dx.wait()

    def _block(j, _):
        for g in range(5):
            off = j * 80 + g * L
            zvec = zbuf[pl.ds(off, L)]
            b0 = plsc.load_gather(t0, [zvec])
            b1 = plsc.load_gather(t1, [zvec])
            nrec = plsc.load_gather(t2, [zvec])
            x = xbuf[pl.ds(off, L)]
            obuf[pl.ds(off, L)] = b0 / (1.0 + jnp.exp((x - b1) * nrec))
        return ()

    lax.fori_loop(0, ROWS // 80, _block, ())
    pltpu.sync_copy(obuf, out_hbm.at[pl.ds(base, ROWS)])


@functools.cache
def _build_kernels():
    mesh = _mesh()
    scatter = pl.kernel(
        _scatter_kernel_body,
        out_type=jax.ShapeDtypeStruct((NC, 4, QR, L), jnp.float32),
        mesh=mesh,
        compiler_params=_PARAMS,
        scratch_types=[
            pltpu.VMEM((ROWS,), jnp.int32),
            pltpu.VMEM((CH, 16), jnp.float32),
            pltpu.VMEM((CH, 16), jnp.float32),
            pltpu.VMEM((QR, L), jnp.float32),
            pltpu.VMEM((QR, L), jnp.float32),
            pltpu.VMEM((QR, L), jnp.float32),
            pltpu.VMEM((QR, L), jnp.float32),
            pltpu.VMEM((QRP,), jnp.int32),
            pltpu.VMEM_SHARED((QR, L), jnp.float32),
            pltpu.VMEM_SHARED((QR, L), jnp.float32),
            pltpu.VMEM_SHARED((QR, L), jnp.float32),
            pltpu.VMEM_SHARED((QR, L), jnp.float32),
            pltpu.SemaphoreType.DMA,
            pltpu.SemaphoreType.DMA,
            pltpu.SemaphoreType.DMA,
        ],
    )
    fused = pl.kernel(
        _fused_kernel_body,
        out_type=jax.ShapeDtypeStruct((N,), jnp.float32),
        mesh=mesh,
        compiler_params=_PARAMS,
        scratch_types=[
            pltpu.VMEM((ROWS,), jnp.int32),
            pltpu.VMEM((ROWS,), jnp.float32),
            pltpu.VMEM((ROWS,), jnp.float32),
            pltpu.VMEM((Q,), jnp.float32),
            pltpu.VMEM((Q,), jnp.float32),
            pltpu.VMEM((Q,), jnp.float32),
            pltpu.VMEM((8, CB_PAD), jnp.float32),
            pltpu.VMEM((CB_PAD,), jnp.float32),
            pltpu.VMEM((CB_PAD,), jnp.float32),
            pltpu.VMEM((CB_PAD,), jnp.float32),
            pltpu.VMEM((L,), jnp.float32),
            pltpu.VMEM_SHARED((3, Q), jnp.float32),
            pltpu.SemaphoreType.DMA,
        ],
    )
    return scatter, fused


def kernel(X_input, Z0, emb0, beta_1, beta_2, beta_3):
    scatter, fused = _build_kernels()
    z = Z0.astype(jnp.int32)
    x = X_input.astype(jnp.float32).reshape(N)
    betavec = jnp.concatenate([
        jnp.stack([beta_1, beta_2, beta_3]).astype(jnp.float32),
        jnp.zeros((13,), jnp.float32),
    ])
    partials = scatter(z, emb0).reshape(NC, 4, Q)
    out = fused(z, x, partials, betavec)
    return out.reshape(N, 1)
